# baseline (speedup 1.0000x reference)
import jax
import jax.numpy as jnp
from jax import lax
from jax.experimental import pallas as pl
from jax.experimental.pallas import tpu as pltpu
from jax.experimental.pallas import tpu_sc as plsc

B = 1024
S = 512
HID = 128

NC = 2
NS = 16
NW = NC * NS
TOK = B * S
CH = 128

K = 8
BC = B // K
TOK_C = TOK // K
PER_WC = TOK_C // NW
NCH_C = PER_WC // CH


NSLOT = 4


def _sc_gather_body(table_hbm, ids_hbm, out_hbm, idx_v, rows_v, gsems, ssems):
    wid = lax.axis_index("s") * NC + lax.axis_index("c")
    base = wid * PER_WC
    pltpu.sync_copy(ids_hbm.at[wid], idx_v)

    def start_gather(j, slot):
        pltpu.async_copy(
            table_hbm.at[idx_v.at[j]], rows_v.at[slot], gsems[slot])

    def wait_gather(slot):
        pltpu.make_async_copy(
            table_hbm.at[idx_v.at[slot]], rows_v.at[slot], gsems[slot]).wait()

    def sdesc(j, slot):
        return pltpu.make_async_copy(
            rows_v.at[slot],
            out_hbm.at[pl.ds(base + j * CH, CH)],
            ssems[slot])

    start_gather(0, 0)
    start_gather(1, 1)

    def outer(m, carry):
        for p in range(NSLOT):
            c = m * NSLOT + p
            wait_gather(p)
            sdesc(c, p).start()
            p2 = (p + 2) % NSLOT

            @pl.when(c >= 2)
            def _wait_old_scatter():
                sdesc(c - 2, p2).wait()

            @pl.when(c + 2 < NCH_C)
            def _start_next_gather():
                start_gather(c + 2, p2)
        return carry

    lax.fori_loop(0, NCH_C // NSLOT, outer, 0)
    sdesc(NCH_C - 2, (NCH_C - 2) % NSLOT).wait()
    sdesc(NCH_C - 1, (NCH_C - 1) % NSLOT).wait()


def _sc_gather(token_table, ids3):
    mesh = plsc.VectorSubcoreMesh(core_axis_name="c", subcore_axis_name="s")
    return pl.kernel(
        _sc_gather_body,
        out_type=jax.ShapeDtypeStruct((TOK_C, HID), jnp.float32),
        mesh=mesh,
        scratch_types=[
            pltpu.VMEM((NCH_C, CH), jnp.int32),
            pltpu.VMEM((NSLOT, CH, HID), jnp.float32),
            [pltpu.SemaphoreType.DMA] * NSLOT,
            [pltpu.SemaphoreType.DMA] * NSLOT,
        ],
    )(token_table, ids3)


RB = 32


def _ln_body(x_ref, tt_ref, pos_ref, ty_ref, w_ref, b_ref, prev_ref, o_ref):
    del prev_ref
    x = x_ref[...]
    tt = tt_ref[...].astype(jnp.float32)
    pos = pos_ref[...]
    t0 = ty_ref[0]
    dt = ty_ref[1] - t0
    e = x + pos[None, :, :] + t0[None, None, :] + tt[:, :, None] * dt[None, None, :]
    mean = jnp.mean(e, axis=-1, keepdims=True)
    var = jnp.mean(jnp.square(e - mean), axis=-1, keepdims=True)
    normed = (e - mean) * lax.rsqrt(var + 1e-5)
    o_ref[...] = normed * w_ref[0][None, None, :] + b_ref[0][None, None, :]


def _ln_body_first(x_ref, tt_ref, pos_ref, ty_ref, w_ref, b_ref, o_ref):
    _ln_body(x_ref, tt_ref, pos_ref, ty_ref, w_ref, b_ref, None, o_ref)


def _tc_ln_chunk(c, x, tt_c, pos_enc, type_table, w2, b2, prev):
    specs = [
        pl.BlockSpec((RB, S, HID), lambda i: (i, 0, 0)),
        pl.BlockSpec((RB, S), lambda i: (i, 0)),
        pl.BlockSpec((S, HID), lambda i: (0, 0)),
        pl.BlockSpec((2, HID), lambda i: (0, 0)),
        pl.BlockSpec((1, HID), lambda i: (0, 0)),
        pl.BlockSpec((1, HID), lambda i: (0, 0)),
    ]
    args = [x, tt_c, pos_enc, type_table, w2, b2]
    body = _ln_body_first
    aliases = {}
    if prev is not None:
        specs.append(pl.BlockSpec(memory_space=pltpu.MemorySpace.HBM))
        args.append(prev)
        body = _ln_body
        aliases = {6: 0}
    return pl.pallas_call(
        body,
        grid=(BC // RB,),
        in_specs=specs,
        out_specs=pl.BlockSpec((RB, S, HID), lambda i, _c=c: (_c * (BC // RB) + i, 0, 0)),
        out_shape=jax.ShapeDtypeStruct((B, S, HID), jnp.float32),
        input_output_aliases=aliases,
    )(*args)


def kernel(input_ids, token_type_ids, token_table, type_table, pos_enc, ln_weight, ln_bias):
    ids4 = input_ids.astype(jnp.int32).reshape(K, NW, NCH_C, CH)
    tt4 = token_type_ids.reshape(K, BC, S)
    w2 = ln_weight.reshape(1, HID)
    b2 = ln_bias.reshape(1, HID)
    gathered = [_sc_gather(token_table, ids4[c]).reshape(BC, S, HID) for c in range(K)]
    out = None
    for c in range(K):
        out = _tc_ln_chunk(c, gathered[c], tt4[c], pos_enc, type_table, w2, b2, out)
    return out


# speedup vs baseline: 1.0222x; 1.0222x over previous
import jax
import jax.numpy as jnp
from jax import lax
from jax.experimental import pallas as pl
from jax.experimental.pallas import tpu as pltpu
from jax.experimental.pallas import tpu_sc as plsc

B = 1024
S = 512
HID = 128

NC = 2
NS = 16
NW = NC * NS
TOK = B * S
CH = 128

K = 8
BC = B // K
TOK_C = TOK // K
PER_WC = TOK_C // NW
NCH_C = PER_WC // CH


NSLOT = 4


def _sc_gather_body(table_hbm, ids_hbm, out_hbm, idx_v, rows_v, gsems, ssems):
    wid = lax.axis_index("s") * NC + lax.axis_index("c")
    base = wid * PER_WC
    pltpu.sync_copy(ids_hbm.at[wid], idx_v)

    def start_gather(j, slot):
        pltpu.async_copy(
            table_hbm.at[idx_v.at[j]], rows_v.at[slot], gsems[slot])

    def wait_gather(slot):
        pltpu.make_async_copy(
            table_hbm.at[idx_v.at[slot]], rows_v.at[slot], gsems[slot]).wait()

    def sdesc(j, slot):
        return pltpu.make_async_copy(
            rows_v.at[slot],
            out_hbm.at[pl.ds(base + j * CH, CH)],
            ssems[slot])

    start_gather(0, 0)
    start_gather(1, 1)

    def outer(m, carry):
        for p in range(NSLOT):
            c = m * NSLOT + p
            wait_gather(p)
            sdesc(c, p).start()
            p2 = (p + 2) % NSLOT

            @pl.when(c >= 2)
            def _wait_old_scatter():
                sdesc(c - 2, p2).wait()

            @pl.when(c + 2 < NCH_C)
            def _start_next_gather():
                start_gather(c + 2, p2)
        return carry

    lax.fori_loop(0, NCH_C // NSLOT, outer, 0)
    sdesc(NCH_C - 2, (NCH_C - 2) % NSLOT).wait()
    sdesc(NCH_C - 1, (NCH_C - 1) % NSLOT).wait()


def _sc_gather(token_table, ids3):
    mesh = plsc.VectorSubcoreMesh(core_axis_name="c", subcore_axis_name="s")
    return pl.kernel(
        _sc_gather_body,
        out_type=jax.ShapeDtypeStruct((TOK_C, HID), jnp.float32),
        mesh=mesh,
        scratch_types=[
            pltpu.VMEM((NCH_C, CH), jnp.int32),
            pltpu.VMEM((NSLOT, CH, HID), jnp.float32),
            [pltpu.SemaphoreType.DMA] * NSLOT,
            [pltpu.SemaphoreType.DMA] * NSLOT,
        ],
    )(token_table, ids3)


RB = 16


def _ln_body(x_ref, tt_ref, pos_ref, ty_ref, w_ref, b_ref, prev_ref, o_ref):
    del prev_ref
    x = x_ref[...]
    tt = tt_ref[...].astype(jnp.float32)
    pos = pos_ref[...]
    t0 = ty_ref[0]
    dt = ty_ref[1] - t0
    e = x + pos[None, :, :] + t0[None, None, :] + tt[:, :, None] * dt[None, None, :]
    mean = jnp.mean(e, axis=-1, keepdims=True)
    var = jnp.mean(jnp.square(e - mean), axis=-1, keepdims=True)
    normed = (e - mean) * lax.rsqrt(var + 1e-5)
    o_ref[...] = normed * w_ref[0][None, None, :] + b_ref[0][None, None, :]


def _ln_body_first(x_ref, tt_ref, pos_ref, ty_ref, w_ref, b_ref, o_ref):
    _ln_body(x_ref, tt_ref, pos_ref, ty_ref, w_ref, b_ref, None, o_ref)


def _tc_ln_chunk(c, x, tt_c, pos_enc, type_table, w2, b2, prev):
    specs = [
        pl.BlockSpec((RB, S, HID), lambda i: (i, 0, 0)),
        pl.BlockSpec((RB, S), lambda i: (i, 0)),
        pl.BlockSpec((S, HID), lambda i: (0, 0)),
        pl.BlockSpec((2, HID), lambda i: (0, 0)),
        pl.BlockSpec((1, HID), lambda i: (0, 0)),
        pl.BlockSpec((1, HID), lambda i: (0, 0)),
    ]
    args = [x, tt_c, pos_enc, type_table, w2, b2]
    body = _ln_body_first
    aliases = {}
    if prev is not None:
        specs.append(pl.BlockSpec(memory_space=pltpu.MemorySpace.HBM))
        args.append(prev)
        body = _ln_body
        aliases = {6: 0}
    return pl.pallas_call(
        body,
        grid=(BC // RB,),
        in_specs=specs,
        out_specs=pl.BlockSpec((RB, S, HID), lambda i, _c=c: (_c * (BC // RB) + i, 0, 0)),
        out_shape=jax.ShapeDtypeStruct((B, S, HID), jnp.float32),
        input_output_aliases=aliases,
    )(*args)


def kernel(input_ids, token_type_ids, token_table, type_table, pos_enc, ln_weight, ln_bias):
    ids4 = input_ids.astype(jnp.int32).reshape(K, NW, NCH_C, CH)
    tt4 = token_type_ids.reshape(K, BC, S)
    w2 = ln_weight.reshape(1, HID)
    b2 = ln_bias.reshape(1, HID)
    gathered = [_sc_gather(token_table, ids4[c]).reshape(BC, S, HID) for c in range(K)]
    out = None
    for c in range(K):
        out = _tc_ln_chunk(c, gathered[c], tt4[c], pos_enc, type_table, w2, b2, out)
    return out
